# baseline (speedup 1.0000x reference)
import functools

import jax
import jax.numpy as jnp
import numpy as np
from jax import lax
from jax.experimental import pallas as pl
from jax.experimental.pallas import tpu as pltpu
from jax.experimental.pallas import tpu_sc as plsc

_N_NODES = 10000
_N_EDGES = 320000
_D_FEAT = 128
_SCALAR_DIM = 64
_R_PREV = 16
_OUT_DIM = _R_PREV + 2 * _SCALAR_DIM

_CHUNK = 128
_KSUB = 4
_SUPER = _KSUB * _CHUNK
_N_SUPERS = _N_EDGES // _SUPER
_NC = 2
_NS = 16
_NW = _NC * _NS
_ITERS = (_N_SUPERS + _NW - 1) // _NW

_INV_SQRT_FAN_IN = np.float32(1.0 / np.sqrt(np.float32(_D_FEAT)))


def _matmul_body(x_ref, w_ref, o_ref):
    o_ref[...] = jax.lax.dot_general(
        x_ref[...], w_ref[...],
        dimension_numbers=(((1,), (0,)), ((), ())),
        preferred_element_type=jnp.float32,
    ) * _INV_SQRT_FAN_IN


_node_linear = pl.pallas_call(
    _matmul_body,
    out_shape=jax.ShapeDtypeStruct((_N_NODES, _SCALAR_DIM), jnp.float32),
)


def _gather_body(feat, esrc, edst, prev, out, isrc_v, idst_v, rsrc_v, rdst_v, sem_in, sem_g, sem_wr):
    wid = lax.axis_index("s") * _NC + lax.axis_index("c")

    def body(s, carry):
        sid = s * _NW + wid

        @pl.when(sid < _N_SUPERS)
        def _():
            r0 = sid * _SUPER
            cprev = pltpu.make_async_copy(
                prev.at[pl.ds(r0, _SUPER), :],
                out.at[pl.ds(r0, _SUPER), pl.ds(0, _R_PREV)],
                sem_wr,
            )
            cprev.start()
            cin = [
                pltpu.make_async_copy(esrc.at[sid], isrc_v, sem_in),
                pltpu.make_async_copy(edst.at[sid], idst_v, sem_in),
            ]
            for c in cin:
                c.start()
            for c in cin:
                c.wait()
            cg = []
            for j in range(_KSUB):
                cg.append(pltpu.make_async_copy(
                    feat.at[isrc_v.at[j]], rsrc_v.at[pl.ds(j * _CHUNK, _CHUNK), :], sem_g))
                cg.append(pltpu.make_async_copy(
                    feat.at[idst_v.at[j]], rdst_v.at[pl.ds(j * _CHUNK, _CHUNK), :], sem_g))
            for c in cg:
                c.start()
            for c in cg:
                c.wait()
            cw = [
                pltpu.make_async_copy(
                    rsrc_v, out.at[pl.ds(r0, _SUPER), pl.ds(_R_PREV, _SCALAR_DIM)], sem_wr),
                pltpu.make_async_copy(
                    rdst_v, out.at[pl.ds(r0, _SUPER), pl.ds(_R_PREV + _SCALAR_DIM, _SCALAR_DIM)], sem_wr),
            ]
            for c in cw:
                c.start()
            for c in cw:
                c.wait()
            cprev.wait()

        return carry

    lax.fori_loop(0, _ITERS, body, 0)


_gather_concat = functools.partial(
    pl.kernel,
    out_type=jax.ShapeDtypeStruct((_N_EDGES, _OUT_DIM), jnp.float32),
    mesh=plsc.VectorSubcoreMesh(
        core_axis_name="c", subcore_axis_name="s", num_cores=_NC, num_subcores=_NS
    ),
    scratch_types=[
        pltpu.VMEM((_KSUB, _CHUNK), jnp.int32),
        pltpu.VMEM((_KSUB, _CHUNK), jnp.int32),
        pltpu.VMEM((_SUPER, _SCALAR_DIM), jnp.float32),
        pltpu.VMEM((_SUPER, _SCALAR_DIM), jnp.float32),
        pltpu.SemaphoreType.DMA,
        pltpu.SemaphoreType.DMA,
        pltpu.SemaphoreType.DMA,
    ],
    compiler_params=pltpu.CompilerParams(use_tc_tiling_on_sc=False),
)(_gather_body)


@jax.jit
def kernel(x, radial_weights_prev, edge_index, W):
    feat = _node_linear(x, W)
    edge_src = edge_index[1].reshape(_N_SUPERS, _KSUB, _CHUNK)
    edge_dst = edge_index[0].reshape(_N_SUPERS, _KSUB, _CHUNK)
    return _gather_concat(feat, edge_src, edge_dst, radial_weights_prev)


# speedup vs baseline: 2.3291x; 2.3291x over previous
import functools

import jax
import jax.numpy as jnp
import numpy as np
from jax import lax
from jax.experimental import pallas as pl
from jax.experimental.pallas import tpu as pltpu
from jax.experimental.pallas import tpu_sc as plsc

_N_NODES = 10000
_N_EDGES = 320000
_D_FEAT = 128
_SCALAR_DIM = 64
_R_PREV = 16
_OUT_DIM = _R_PREV + 2 * _SCALAR_DIM

_CHUNK = 128
_KSUB = 4
_SUPER = _KSUB * _CHUNK
_N_SUPERS = _N_EDGES // _SUPER
_NC = 2
_NS = 16
_NW = _NC * _NS
_ITERS = (_N_SUPERS + _NW - 1) // _NW

_INV_SQRT_FAN_IN = np.float32(1.0 / np.sqrt(np.float32(_D_FEAT)))


def _matmul_body(x_ref, w_ref, o_ref):
    o_ref[...] = jax.lax.dot_general(
        x_ref[...], w_ref[...],
        dimension_numbers=(((1,), (0,)), ((), ())),
        preferred_element_type=jnp.float32,
    ) * _INV_SQRT_FAN_IN


_node_linear = pl.pallas_call(
    _matmul_body,
    out_shape=jax.ShapeDtypeStruct((_N_NODES, _SCALAR_DIM), jnp.float32),
)


def _gather_body(feat, esrc, edst, prev, out, isrc_v, idst_v, rsrc_v, rdst_v, prev_v, sem_idx, sem_prev, sem_g, sem_wr):
    wid = lax.axis_index("s") * _NC + lax.axis_index("c")

    def body(s, carry):
        sid = s * _NW + wid

        @pl.when(sid < _N_SUPERS)
        def _():
            r0 = sid * _SUPER
            cin = [
                pltpu.make_async_copy(esrc.at[sid], isrc_v, sem_idx),
                pltpu.make_async_copy(edst.at[sid], idst_v, sem_idx),
            ]
            for c in cin:
                c.start()
            cprev_in = pltpu.make_async_copy(
                prev.at[pl.ds(r0, _SUPER), :], prev_v, sem_prev)
            cprev_in.start()
            for c in cin:
                c.wait()
            cg = []
            for j in range(_KSUB):
                cg.append(pltpu.make_async_copy(
                    feat.at[isrc_v.at[j]], rsrc_v.at[pl.ds(j * _CHUNK, _CHUNK), :], sem_g))
                cg.append(pltpu.make_async_copy(
                    feat.at[idst_v.at[j]], rdst_v.at[pl.ds(j * _CHUNK, _CHUNK), :], sem_g))
            for c in cg:
                c.start()
            for c in cg:
                c.wait()
            cprev_in.wait()
            cw = [
                pltpu.make_async_copy(
                    prev_v, out.at[pl.ds(r0, _SUPER), pl.ds(0, _R_PREV)], sem_wr),
                pltpu.make_async_copy(
                    rsrc_v, out.at[pl.ds(r0, _SUPER), pl.ds(_R_PREV, _SCALAR_DIM)], sem_wr),
                pltpu.make_async_copy(
                    rdst_v, out.at[pl.ds(r0, _SUPER), pl.ds(_R_PREV + _SCALAR_DIM, _SCALAR_DIM)], sem_wr),
            ]
            for c in cw:
                c.start()
            for c in cw:
                c.wait()

        return carry

    lax.fori_loop(0, _ITERS, body, 0)


_gather_concat = functools.partial(
    pl.kernel,
    out_type=jax.ShapeDtypeStruct((_N_EDGES, _OUT_DIM), jnp.float32),
    mesh=plsc.VectorSubcoreMesh(
        core_axis_name="c", subcore_axis_name="s", num_cores=_NC, num_subcores=_NS
    ),
    scratch_types=[
        pltpu.VMEM((_KSUB, _CHUNK), jnp.int32),
        pltpu.VMEM((_KSUB, _CHUNK), jnp.int32),
        pltpu.VMEM((_SUPER, _SCALAR_DIM), jnp.float32),
        pltpu.VMEM((_SUPER, _SCALAR_DIM), jnp.float32),
        pltpu.VMEM((_SUPER, _R_PREV), jnp.float32),
        pltpu.SemaphoreType.DMA,
        pltpu.SemaphoreType.DMA,
        pltpu.SemaphoreType.DMA,
        pltpu.SemaphoreType.DMA,
    ],
    compiler_params=pltpu.CompilerParams(use_tc_tiling_on_sc=False),
)(_gather_body)


@jax.jit
def kernel(x, radial_weights_prev, edge_index, W):
    feat = _node_linear(x, W)
    edge_src = edge_index[1].reshape(_N_SUPERS, _KSUB, _CHUNK)
    edge_dst = edge_index[0].reshape(_N_SUPERS, _KSUB, _CHUNK)
    return _gather_concat(feat, edge_src, edge_dst, radial_weights_prev)


# speedup vs baseline: 2.8398x; 1.2193x over previous
import functools

import jax
import jax.numpy as jnp
import numpy as np
from jax import lax
from jax.experimental import pallas as pl
from jax.experimental.pallas import tpu as pltpu
from jax.experimental.pallas import tpu_sc as plsc

_N_NODES = 10000
_N_EDGES = 320000
_D_FEAT = 128
_SCALAR_DIM = 64
_R_PREV = 16
_OUT_DIM = _R_PREV + 2 * _SCALAR_DIM

_CHUNK = 128
_KSUB = 4
_SUPER = _KSUB * _CHUNK
_N_SUPERS = _N_EDGES // _SUPER
_NC = 2
_NS = 16
_NW = _NC * _NS
_ITERS = (_N_SUPERS + _NW - 1) // _NW

_INV_SQRT_FAN_IN = np.float32(1.0 / np.sqrt(np.float32(_D_FEAT)))


def _matmul_body(x_ref, w_ref, o_ref):
    o_ref[...] = jax.lax.dot_general(
        x_ref[...], w_ref[...],
        dimension_numbers=(((1,), (0,)), ((), ())),
        preferred_element_type=jnp.float32,
    ) * _INV_SQRT_FAN_IN


_node_linear = pl.pallas_call(
    _matmul_body,
    out_shape=jax.ShapeDtypeStruct((_N_NODES, _SCALAR_DIM), jnp.float32),
)


def _gather_body(feat, esrc, edst, gboth, isrc_v, idst_v, rsrc_v, rdst_v, sem_idx, sem_g, sem_wr):
    wid = lax.axis_index("s") * _NC + lax.axis_index("c")

    def body(s, carry):
        sid = s * _NW + wid

        @pl.when(sid < _N_SUPERS)
        def _():
            r0 = sid * _SUPER
            cin = []
            for j in range(_KSUB):
                cin.append(pltpu.make_async_copy(
                    esrc.at[pl.ds(r0 + j * _CHUNK, _CHUNK)], isrc_v.at[j], sem_idx))
                cin.append(pltpu.make_async_copy(
                    edst.at[pl.ds(r0 + j * _CHUNK, _CHUNK)], idst_v.at[j], sem_idx))
            for c in cin:
                c.start()
            for c in cin:
                c.wait()
            cg = []
            for j in range(_KSUB):
                rows = pl.ds(j * _CHUNK, _CHUNK)
                cg.append(pltpu.make_async_copy(
                    feat.at[isrc_v.at[j]], rsrc_v.at[rows, :], sem_g))
                cg.append(pltpu.make_async_copy(
                    feat.at[idst_v.at[j]], rdst_v.at[rows, :], sem_g))
            for c in cg:
                c.start()
            for c in cg:
                c.wait()
            cw = [
                pltpu.make_async_copy(
                    rsrc_v, gboth.at[pl.ds(r0, _SUPER), pl.ds(0, _SCALAR_DIM)], sem_wr),
                pltpu.make_async_copy(
                    rdst_v, gboth.at[pl.ds(r0, _SUPER), pl.ds(_SCALAR_DIM, _SCALAR_DIM)], sem_wr),
            ]
            for c in cw:
                c.start()
            for c in cw:
                c.wait()

        return carry

    lax.fori_loop(0, _ITERS, body, 0)


_gather_both = functools.partial(
    pl.kernel,
    out_type=jax.ShapeDtypeStruct((_N_EDGES, 2 * _SCALAR_DIM), jnp.float32),
    mesh=plsc.VectorSubcoreMesh(
        core_axis_name="c", subcore_axis_name="s", num_cores=_NC, num_subcores=_NS
    ),
    scratch_types=[
        pltpu.VMEM((_KSUB, _CHUNK), jnp.int32),
        pltpu.VMEM((_KSUB, _CHUNK), jnp.int32),
        pltpu.VMEM((_SUPER, _SCALAR_DIM), jnp.float32),
        pltpu.VMEM((_SUPER, _SCALAR_DIM), jnp.float32),
        pltpu.SemaphoreType.DMA,
        pltpu.SemaphoreType.DMA,
        pltpu.SemaphoreType.DMA,
    ],
    compiler_params=pltpu.CompilerParams(use_tc_tiling_on_sc=False),
)(_gather_body)


_CB = 8000


def _concat_body(p_ref, b_ref, o_ref):
    o_ref[...] = jnp.concatenate([p_ref[...], b_ref[...]], axis=-1)


_concat2 = pl.pallas_call(
    _concat_body,
    grid=(_N_EDGES // _CB,),
    in_specs=[
        pl.BlockSpec((_CB, _R_PREV), lambda i: (i, 0)),
        pl.BlockSpec((_CB, 2 * _SCALAR_DIM), lambda i: (i, 0)),
    ],
    out_specs=pl.BlockSpec((_CB, _OUT_DIM), lambda i: (i, 0)),
    out_shape=jax.ShapeDtypeStruct((_N_EDGES, _OUT_DIM), jnp.float32),
)


@jax.jit
def kernel(x, radial_weights_prev, edge_index, W):
    feat = _node_linear(x, W)
    edge_src = edge_index[1]
    edge_dst = edge_index[0]
    gboth = _gather_both(feat, edge_src, edge_dst)
    return _concat2(radial_weights_prev, gboth)


# speedup vs baseline: 2.8495x; 1.0034x over previous
import functools

import jax
import jax.numpy as jnp
import numpy as np
from jax import lax
from jax.experimental import pallas as pl
from jax.experimental.pallas import tpu as pltpu
from jax.experimental.pallas import tpu_sc as plsc

_N_NODES = 10000
_N_EDGES = 320000
_D_FEAT = 128
_SCALAR_DIM = 64
_R_PREV = 16
_OUT_DIM = _R_PREV + 2 * _SCALAR_DIM

_CHUNK = 128
_KSUB = 4
_SUPER = _KSUB * _CHUNK
_NC = 2
_NS = 16
_NW = _NC * _NS

_NB = 5
_EB = _N_EDGES // _NB
_SUPERS_B = _EB // _SUPER
_ITERS_B = (_SUPERS_B + _NW - 1) // _NW

_INV_SQRT_FAN_IN = np.float32(1.0 / np.sqrt(np.float32(_D_FEAT)))


def _matmul_body(x_ref, w_ref, o_ref):
    o_ref[...] = jax.lax.dot_general(
        x_ref[...], w_ref[...],
        dimension_numbers=(((1,), (0,)), ((), ())),
        preferred_element_type=jnp.float32,
    ) * _INV_SQRT_FAN_IN


_node_linear = pl.pallas_call(
    _matmul_body,
    out_shape=jax.ShapeDtypeStruct((_N_NODES, _SCALAR_DIM), jnp.float32),
)


def _gather_body(base, feat, esrc, edst, gb, isrc_v, idst_v, rsrc_v, rdst_v,
                 sem_idx, sem_g, sem_wr):
    wid = lax.axis_index("s") * _NC + lax.axis_index("c")

    def body(s, carry):
        sid = s * _NW + wid

        @pl.when(sid < _SUPERS_B)
        def _():
            r0 = sid * _SUPER
            e0 = base + r0
            cin = []
            for j in range(_KSUB):
                cin.append(pltpu.make_async_copy(
                    esrc.at[pl.ds(e0 + j * _CHUNK, _CHUNK)], isrc_v.at[j], sem_idx))
                cin.append(pltpu.make_async_copy(
                    edst.at[pl.ds(e0 + j * _CHUNK, _CHUNK)], idst_v.at[j], sem_idx))
            for c in cin:
                c.start()
            for c in cin:
                c.wait()
            cg = []
            for j in range(_KSUB):
                rows = pl.ds(j * _CHUNK, _CHUNK)
                cg.append(pltpu.make_async_copy(
                    feat.at[isrc_v.at[j]], rsrc_v.at[rows, :], sem_g))
                cg.append(pltpu.make_async_copy(
                    feat.at[idst_v.at[j]], rdst_v.at[rows, :], sem_g))
            for c in cg:
                c.start()
            for c in cg:
                c.wait()
            cw = [
                pltpu.make_async_copy(
                    rsrc_v, gb.at[pl.ds(r0, _SUPER), pl.ds(0, _SCALAR_DIM)], sem_wr),
                pltpu.make_async_copy(
                    rdst_v, gb.at[pl.ds(r0, _SUPER), pl.ds(_SCALAR_DIM, _SCALAR_DIM)], sem_wr),
            ]
            for c in cw:
                c.start()
            for c in cw:
                c.wait()

        return carry

    lax.fori_loop(0, _ITERS_B, body, 0)


def _make_gather(b):
    return functools.partial(
        pl.kernel,
        out_type=jax.ShapeDtypeStruct((_EB, 2 * _SCALAR_DIM), jnp.float32),
        mesh=plsc.VectorSubcoreMesh(
            core_axis_name="c", subcore_axis_name="s", num_cores=_NC, num_subcores=_NS
        ),
        scratch_types=[
            pltpu.VMEM((_KSUB, _CHUNK), jnp.int32),
            pltpu.VMEM((_KSUB, _CHUNK), jnp.int32),
            pltpu.VMEM((_SUPER, _SCALAR_DIM), jnp.float32),
            pltpu.VMEM((_SUPER, _SCALAR_DIM), jnp.float32),
            pltpu.SemaphoreType.DMA,
            pltpu.SemaphoreType.DMA,
            pltpu.SemaphoreType.DMA,
        ],
        compiler_params=pltpu.CompilerParams(use_tc_tiling_on_sc=False),
    )(functools.partial(_gather_body, b * _EB))


_gathers = [_make_gather(b) for b in range(_NB)]


_CB = 8000
_GSTEPS = _EB // _CB


def _concat_body(p_ref, b_ref, o_ref):
    o_ref[...] = jnp.concatenate([p_ref[...], b_ref[...]], axis=-1)


def _concat_body_alias(a_ref, p_ref, b_ref, o_ref):
    del a_ref
    o_ref[...] = jnp.concatenate([p_ref[...], b_ref[...]], axis=-1)


def _make_concat(b):
    base = b * _GSTEPS
    if b == 0:
        return pl.pallas_call(
            _concat_body,
            grid=(_GSTEPS,),
            in_specs=[
                pl.BlockSpec((_CB, _R_PREV), lambda i: (base + i, 0)),
                pl.BlockSpec((_CB, 2 * _SCALAR_DIM), lambda i: (i, 0)),
            ],
            out_specs=pl.BlockSpec((_CB, _OUT_DIM), lambda i: (base + i, 0)),
            out_shape=jax.ShapeDtypeStruct((_N_EDGES, _OUT_DIM), jnp.float32),
        )
    return pl.pallas_call(
        _concat_body_alias,
        grid=(_GSTEPS,),
        in_specs=[
            pl.BlockSpec(memory_space=pl.ANY),
            pl.BlockSpec((_CB, _R_PREV), lambda i: (base + i, 0)),
            pl.BlockSpec((_CB, 2 * _SCALAR_DIM), lambda i: (i, 0)),
        ],
        out_specs=pl.BlockSpec((_CB, _OUT_DIM), lambda i: (base + i, 0)),
        out_shape=jax.ShapeDtypeStruct((_N_EDGES, _OUT_DIM), jnp.float32),
        input_output_aliases={0: 0},
    )


_concats = [_make_concat(b) for b in range(_NB)]


@jax.jit
def kernel(x, radial_weights_prev, edge_index, W):
    feat = _node_linear(x, W)
    edge_src = edge_index[1]
    edge_dst = edge_index[0]
    gbs = [_gathers[b](feat, edge_src, edge_dst) for b in range(_NB)]
    out = _concats[0](radial_weights_prev, gbs[0])
    for b in range(1, _NB):
        out = _concats[b](out, radial_weights_prev, gbs[b])
    return out


# speedup vs baseline: 6.1254x; 2.1496x over previous
import functools

import jax
import jax.numpy as jnp
import numpy as np
from jax import lax
from jax.experimental import pallas as pl
from jax.experimental.pallas import tpu as pltpu
from jax.experimental.pallas import tpu_sc as plsc

_N_NODES = 10000
_N_EDGES = 320000
_D_FEAT = 128
_SCALAR_DIM = 64
_R_PREV = 16
_OUT_DIM = _R_PREV + 2 * _SCALAR_DIM

_CHUNK = 128
_KSUB = 4
_SUPER = _KSUB * _CHUNK
_NC = 2
_NS = 16
_NW = _NC * _NS

_NB = 5
_EB = _N_EDGES // _NB
_SUPERS_B = _EB // _SUPER
_ITERS_B = (_SUPERS_B + _NW - 1) // _NW

_INV_SQRT_FAN_IN = np.float32(1.0 / np.sqrt(np.float32(_D_FEAT)))


def _matmul_body(x_ref, w_ref, o_ref):
    o_ref[...] = jax.lax.dot_general(
        x_ref[...], w_ref[...],
        dimension_numbers=(((1,), (0,)), ((), ())),
        preferred_element_type=jnp.float32,
    ) * _INV_SQRT_FAN_IN


_node_linear = pl.pallas_call(
    _matmul_body,
    out_shape=jax.ShapeDtypeStruct((_N_NODES, _SCALAR_DIM), jnp.float32),
)


def _gather_body(base, feat, esrc, edst, gb, isrc_v, idst_v, rsrc_v, rdst_v,
                 sem_idx, sem_g, sem_wr):
    wid = lax.axis_index("s") * _NC + lax.axis_index("c")

    def body(s, carry):
        sid = s * _NW + wid

        @pl.when(sid < _SUPERS_B)
        def _():
            r0 = sid * _SUPER
            e0 = base + r0
            cin = []
            for j in range(_KSUB):
                cin.append(pltpu.make_async_copy(
                    esrc.at[pl.ds(e0 + j * _CHUNK, _CHUNK)], isrc_v.at[j], sem_idx))
                cin.append(pltpu.make_async_copy(
                    edst.at[pl.ds(e0 + j * _CHUNK, _CHUNK)], idst_v.at[j], sem_idx))
            for c in cin:
                c.start()
            for c in cin:
                c.wait()
            cg = []
            for j in range(_KSUB):
                rows = pl.ds(j * _CHUNK, _CHUNK)
                cg.append(pltpu.make_async_copy(
                    feat.at[isrc_v.at[j]], rsrc_v.at[rows, :], sem_g))
                cg.append(pltpu.make_async_copy(
                    feat.at[idst_v.at[j]], rdst_v.at[rows, :], sem_g))
            for c in cg:
                c.start()
            for c in cg:
                c.wait()
            cw = [
                pltpu.make_async_copy(
                    rsrc_v, gb.at[pl.ds(r0, _SUPER), pl.ds(0, _SCALAR_DIM)], sem_wr),
                pltpu.make_async_copy(
                    rdst_v, gb.at[pl.ds(r0, _SUPER), pl.ds(_SCALAR_DIM, _SCALAR_DIM)], sem_wr),
            ]
            for c in cw:
                c.start()
            for c in cw:
                c.wait()

        return carry

    lax.fori_loop(0, _ITERS_B, body, 0)


def _make_gather(b):
    return functools.partial(
        pl.kernel,
        out_type=jax.ShapeDtypeStruct((_EB, 2 * _SCALAR_DIM), jnp.float32),
        mesh=plsc.VectorSubcoreMesh(
            core_axis_name="c", subcore_axis_name="s", num_cores=_NC, num_subcores=_NS
        ),
        scratch_types=[
            pltpu.VMEM((_KSUB, _CHUNK), jnp.int32),
            pltpu.VMEM((_KSUB, _CHUNK), jnp.int32),
            pltpu.VMEM((_SUPER, _SCALAR_DIM), jnp.float32),
            pltpu.VMEM((_SUPER, _SCALAR_DIM), jnp.float32),
            pltpu.SemaphoreType.DMA,
            pltpu.SemaphoreType.DMA,
            pltpu.SemaphoreType.DMA,
        ],
        compiler_params=pltpu.CompilerParams(use_tc_tiling_on_sc=False),
    )(functools.partial(_gather_body, b * _EB))


_gathers = [_make_gather(b) for b in range(_NB)]


_CB = 3200
_GSTEPS = _EB // _CB


def _concat_body(p_ref, b_ref, o_ref):
    o_ref[0:_R_PREV, :] = p_ref[...]
    o_ref[_R_PREV:_OUT_DIM, :] = b_ref[...].T


def _concat_body_alias(a_ref, p_ref, b_ref, o_ref):
    del a_ref
    o_ref[0:_R_PREV, :] = p_ref[...]
    o_ref[_R_PREV:_OUT_DIM, :] = b_ref[...].T


def _make_concat(b):
    base = b * _GSTEPS
    if b == 0:
        return pl.pallas_call(
            _concat_body,
            grid=(_GSTEPS,),
            in_specs=[
                pl.BlockSpec((_R_PREV, _CB), lambda i: (0, base + i)),
                pl.BlockSpec((_CB, 2 * _SCALAR_DIM), lambda i: (i, 0)),
            ],
            out_specs=pl.BlockSpec((_OUT_DIM, _CB), lambda i: (0, base + i)),
            out_shape=jax.ShapeDtypeStruct((_OUT_DIM, _N_EDGES), jnp.float32),
        )
    return pl.pallas_call(
        _concat_body_alias,
        grid=(_GSTEPS,),
        in_specs=[
            pl.BlockSpec(memory_space=pl.ANY),
            pl.BlockSpec((_R_PREV, _CB), lambda i: (0, base + i)),
            pl.BlockSpec((_CB, 2 * _SCALAR_DIM), lambda i: (i, 0)),
        ],
        out_specs=pl.BlockSpec((_OUT_DIM, _CB), lambda i: (0, base + i)),
        out_shape=jax.ShapeDtypeStruct((_OUT_DIM, _N_EDGES), jnp.float32),
        input_output_aliases={0: 0},
    )


_concats = [_make_concat(b) for b in range(_NB)]


@jax.jit
def kernel(x, radial_weights_prev, edge_index, W):
    feat = _node_linear(x, W)
    edge_src = edge_index[1]
    edge_dst = edge_index[0]
    prev_t = radial_weights_prev.T
    gbs = [_gathers[b](feat, edge_src, edge_dst) for b in range(_NB)]
    out_t = _concats[0](prev_t, gbs[0])
    for b in range(1, _NB):
        out_t = _concats[b](out_t, prev_t, gbs[b])
    return out_t.T
